# Initial kernel scaffold; baseline (speedup 1.0000x reference)
import jax
import jax.numpy as jnp
from jax.experimental import pallas as pl


def kernel(x, edge_index, W1_l, b1, W1_r, W2_l, b2, W2_r):
    raise NotImplementedError("write your pallas kernel here")


# speedup vs baseline: 16.1627x; 16.1627x over previous
import functools

import jax
import jax.numpy as jnp
from jax import lax
from jax.experimental import pallas as pl
from jax.experimental.pallas import tpu as pltpu
from jax.experimental.pallas import tpu_sc as plsc

N_NODES = 10000
N_EDGES = 320000
D_IN = 128
D_HID = 32
D_OUT = 128

NC = 2
NS = 16
NW = NC * NS
EPW = N_EDGES // NW
CH = 80
NCHUNK = EPW // CH
N_PAD = 10240
RPT = N_PAD // NS

W_AUG = 40
BM = 2000


def _sc_edge_agg(width):
  mesh = plsc.VectorSubcoreMesh(core_axis_name="c", subcore_axis_name="s")

  @functools.partial(
      pl.kernel,
      out_type=jax.ShapeDtypeStruct((NC, N_PAD, width), jnp.float32),
      mesh=mesh,
      compiler_params=pltpu.CompilerParams(use_tc_tiling_on_sc=False),
      scratch_types=[
          pltpu.VMEM_SHARED((N_PAD, width), jnp.float32),
          pltpu.VMEM((NCHUNK, CH), jnp.int32),
          pltpu.VMEM((NCHUNK, CH), jnp.int32),
          pltpu.VMEM((2, CH, width), jnp.float32),
          pltpu.VMEM((RPT, width), jnp.float32),
          pltpu.SemaphoreType.DMA,
          pltpu.SemaphoreType.DMA,
      ],
  )
  def k(table_hbm, src_hbm, dst_hbm, zeros_hbm, out_hbm,
        acc_sh, src_v, dst_v, rows_v, zb_v, sem0, sem1):
    c = lax.axis_index("c")
    s = lax.axis_index("s")
    wid = s * NC + c

    pltpu.sync_copy(zeros_hbm, zb_v)
    pltpu.sync_copy(zb_v, acc_sh.at[pl.ds(s * RPT, RPT)])

    pltpu.sync_copy(src_hbm.at[wid], src_v)
    pltpu.sync_copy(dst_hbm.at[wid], dst_v)
    plsc.subcore_barrier()

    sems = (sem0, sem1)

    def start(j, slot):
      pltpu.async_copy(table_hbm.at[src_v.at[j]], rows_v.at[slot],
                       sems[slot])

    def finish(j, slot):
      pltpu.make_async_copy(table_hbm.at[src_v.at[j]], rows_v.at[slot],
                            sems[slot]).wait()
      pltpu.sync_copy(rows_v.at[slot], acc_sh.at[dst_v.at[j]], add=True)

    start(0, 0)

    def body(jj, carry):
      j0 = jj * 2
      start(j0 + 1, 1)
      finish(j0, 0)
      start(j0 + 2, 0)
      finish(j0 + 1, 1)
      return carry

    lax.fori_loop(0, (NCHUNK - 1) // 2, body, 0)
    finish(NCHUNK - 1, 0)

    plsc.subcore_barrier()
    pltpu.sync_copy(acc_sh.at[pl.ds(s * RPT, RPT)],
                    out_hbm.at[c, pl.ds(s * RPT, RPT)])

  return k


_agg_aug = _sc_edge_agg(W_AUG)
_agg_hid = _sc_edge_agg(D_HID)


def _dense1_body(x_ref, w1l_ref, w1r_ref, b1_ref, pa_ref, r1_ref):
  xb = x_ref[...]
  p = jnp.dot(xb, w1l_ref[...], preferred_element_type=jnp.float32)
  pa_ref[...] = jnp.concatenate(
      [p, jnp.ones((BM, W_AUG - D_HID), jnp.float32)], axis=1)
  r1_ref[...] = (jnp.dot(xb, w1r_ref[...], preferred_element_type=jnp.float32)
                 + b1_ref[...])


def _dense1(x, w1l, w1r, b1):
  return pl.pallas_call(
      _dense1_body,
      grid=(N_NODES // BM,),
      in_specs=[
          pl.BlockSpec((BM, D_IN), lambda i: (i, 0)),
          pl.BlockSpec((D_IN, D_HID), lambda i: (0, 0)),
          pl.BlockSpec((D_IN, D_HID), lambda i: (0, 0)),
          pl.BlockSpec((1, D_HID), lambda i: (0, 0)),
      ],
      out_specs=[
          pl.BlockSpec((BM, W_AUG), lambda i: (i, 0)),
          pl.BlockSpec((BM, D_HID), lambda i: (i, 0)),
      ],
      out_shape=[
          jax.ShapeDtypeStruct((N_NODES, W_AUG), jnp.float32),
          jax.ShapeDtypeStruct((N_NODES, D_HID), jnp.float32),
      ],
  )(x, w1l, w1r, b1)


def _mid_body(a_ref, r1_ref, h_ref, dinv_ref):
  sall = a_ref[0] + a_ref[1]
  deg = sall[:, D_HID:D_HID + 1]
  dinv = 1.0 / jnp.maximum(deg, 1.0)
  h_ref[...] = jnp.maximum(sall[:, 0:D_HID] * dinv + r1_ref[...], 0.0)
  dinv_ref[...] = jnp.broadcast_to(dinv, (BM, 8))


def _mid(agg1, r1):
  return pl.pallas_call(
      _mid_body,
      grid=(N_NODES // BM,),
      in_specs=[
          pl.BlockSpec((NC, BM, W_AUG), lambda i: (0, i, 0)),
          pl.BlockSpec((BM, D_HID), lambda i: (i, 0)),
      ],
      out_specs=[
          pl.BlockSpec((BM, D_HID), lambda i: (i, 0)),
          pl.BlockSpec((BM, 8), lambda i: (i, 0)),
      ],
      out_shape=[
          jax.ShapeDtypeStruct((N_NODES, D_HID), jnp.float32),
          jax.ShapeDtypeStruct((N_NODES, 8), jnp.float32),
      ],
  )(agg1, r1)


def _final_body(a_ref, d_ref, h_ref, w2l_ref, w2r_ref, b2_ref, o_ref):
  s2 = (a_ref[0] + a_ref[1]) * d_ref[:, 0:1]
  o_ref[...] = (jnp.dot(s2, w2l_ref[...], preferred_element_type=jnp.float32)
                + jnp.dot(h_ref[...], w2r_ref[...],
                          preferred_element_type=jnp.float32)
                + b2_ref[...])


def _final(agg2, dinv8, h, w2l, w2r, b2):
  return pl.pallas_call(
      _final_body,
      grid=(N_NODES // BM,),
      in_specs=[
          pl.BlockSpec((NC, BM, D_HID), lambda i: (0, i, 0)),
          pl.BlockSpec((BM, 8), lambda i: (i, 0)),
          pl.BlockSpec((BM, D_HID), lambda i: (i, 0)),
          pl.BlockSpec((D_HID, D_OUT), lambda i: (0, 0)),
          pl.BlockSpec((D_HID, D_OUT), lambda i: (0, 0)),
          pl.BlockSpec((1, D_OUT), lambda i: (0, 0)),
      ],
      out_specs=pl.BlockSpec((BM, D_OUT), lambda i: (i, 0)),
      out_shape=jax.ShapeDtypeStruct((N_NODES, D_OUT), jnp.float32),
  )(agg2, dinv8, h, w2l, w2r, b2)


def kernel(x, edge_index, W1_l, b1, W1_r, W2_l, b2, W2_r):
  ei = edge_index.astype(jnp.int32)
  src = ei[0].reshape(NW, NCHUNK, CH)
  dst = ei[1].reshape(NW, NCHUNK, CH)
  zeros40 = jnp.zeros((RPT, W_AUG), jnp.float32)
  zeros32 = jnp.zeros((RPT, D_HID), jnp.float32)

  p1aug, r1 = _dense1(x, W1_l, W1_r, b1.reshape(1, D_HID))
  agg1 = _agg_aug(p1aug, src, dst, zeros40)
  h, dinv8 = _mid(agg1, r1)
  agg2 = _agg_hid(h, src, dst, zeros32)
  return _final(agg2, dinv8, h, W2_l, W2_r, b2.reshape(1, D_OUT))
